# baseline (speedup 1.0000x reference)
import jax
import jax.numpy as jnp
from jax.experimental import pallas as pl
from jax.experimental.pallas import tpu as pltpu

H = 16
LOOKBACK = 3
BQ = 256


def _kv_proj_body(kv_ref, wkv_ref, out_ref):
    out_ref[...] = jnp.dot(kv_ref[...], wkv_ref[...],
                           preferred_element_type=jnp.float32)


def _attn_body(seg_ref, q_ref, k_ref, v_ref, wq_ref, wo_ref, out_ref):
    qh = jnp.dot(q_ref[0], wq_ref[...], preferred_element_type=jnp.float32)
    bq, d = qh.shape
    dh = d // H
    lkv = k_ref.shape[1]
    scale = dh ** -0.5
    k = k_ref[0]
    v = v_ref[0]
    seg = seg_ref[0].reshape(bq, 1)
    col = jax.lax.broadcasted_iota(jnp.int32, (bq, lkv), 1).astype(jnp.float32)
    mask = (col <= seg) & (col >= seg - LOOKBACK)
    outs = []
    for h in range(H):
        qh_h = qh[:, h * dh:(h + 1) * dh]
        k_h = k[:, h * dh:(h + 1) * dh]
        v_h = v[:, h * dh:(h + 1) * dh]
        s = jax.lax.dot_general(qh_h, k_h, (((1,), (1,)), ((), ())),
                                preferred_element_type=jnp.float32) * scale
        s = jnp.where(mask, s, -jnp.inf)
        m = jnp.max(s, axis=1, keepdims=True)
        e = jnp.exp(s - m)
        p = e / jnp.sum(e, axis=1, keepdims=True)
        outs.append(jnp.dot(p, v_h, preferred_element_type=jnp.float32))
    attn = jnp.concatenate(outs, axis=1).astype(wo_ref.dtype)
    out_ref[0] = jnp.dot(attn, wo_ref[...], preferred_element_type=jnp.float32)


def kernel(q, kv_src, seg_id, Wq, Wkv, Wo):
    b, lq, qdim = q.shape
    lkv = kv_src.shape[1]
    d = Wq.shape[1]
    nq = lq // BQ

    kvp = pl.pallas_call(
        _kv_proj_body,
        out_shape=jax.ShapeDtypeStruct((b * lkv, 2 * d), jnp.float32),
    )(kv_src.reshape(b * lkv, -1), Wkv)
    k = kvp[:, :d].reshape(b, lkv, d)
    v = kvp[:, d:].reshape(b, lkv, d)

    seg = seg_id.astype(jnp.float32).reshape(b * nq, 1, BQ)
    qb = q.astype(jnp.bfloat16)
    Wqb = Wq.astype(jnp.bfloat16)
    Wob = Wo.astype(jnp.bfloat16)

    out = pl.pallas_call(
        _attn_body,
        grid=(b, nq),
        in_specs=[
            pl.BlockSpec((1, 1, BQ), lambda bi, i: (bi * (lq // BQ) + i, 0, 0)),
            pl.BlockSpec((1, BQ, qdim), lambda bi, i: (bi, i, 0)),
            pl.BlockSpec((1, lkv, d), lambda bi, i: (bi, 0, 0)),
            pl.BlockSpec((1, lkv, d), lambda bi, i: (bi, 0, 0)),
            pl.BlockSpec((qdim, d), lambda bi, i: (0, 0)),
            pl.BlockSpec((d, qdim), lambda bi, i: (0, 0)),
        ],
        out_specs=pl.BlockSpec((1, BQ, qdim), lambda bi, i: (bi, i, 0)),
        out_shape=jax.ShapeDtypeStruct((b, lq, qdim), jnp.float32),
    )(seg, qb, k, v, Wqb, Wob)
    return out


# speedup vs baseline: 1.1414x; 1.1414x over previous
import jax
import jax.numpy as jnp
from jax.experimental import pallas as pl
from jax.experimental.pallas import tpu as pltpu

H = 16
LOOKBACK = 3
BQ = 256


def _kv_proj_body(kv_ref, wkv_ref, out_ref):
    out_ref[...] = jnp.dot(kv_ref[...], wkv_ref[...],
                           preferred_element_type=jnp.float32)


def _attn_body(seg_ref, q_ref, k_ref, v_ref, wq_ref, wo_ref, out_ref):
    qh = jnp.dot(q_ref[0].astype(wq_ref.dtype), wq_ref[...],
                 preferred_element_type=jnp.float32)
    bq, d = qh.shape
    dh = d // H
    lkv = k_ref.shape[1]
    scale = dh ** -0.5
    k = k_ref[0]
    v = v_ref[0]
    seg = seg_ref[0].reshape(bq, 1)
    col = jax.lax.broadcasted_iota(jnp.int32, (bq, lkv), 1).astype(jnp.float32)
    mask = (col <= seg) & (col >= seg - LOOKBACK)
    outs = []
    for h in range(H):
        qh_h = qh[:, h * dh:(h + 1) * dh]
        k_h = k[:, h * dh:(h + 1) * dh]
        v_h = v[:, h * dh:(h + 1) * dh]
        s = jax.lax.dot_general(qh_h, k_h, (((1,), (1,)), ((), ())),
                                preferred_element_type=jnp.float32) * scale
        s = jnp.where(mask, s, -jnp.inf)
        m = jnp.max(s, axis=1, keepdims=True)
        e = jnp.exp(s - m)
        p = e / jnp.sum(e, axis=1, keepdims=True)
        outs.append(jnp.dot(p, v_h, preferred_element_type=jnp.float32))
    attn = jnp.concatenate(outs, axis=1).astype(wo_ref.dtype)
    out_ref[0] = jnp.dot(attn, wo_ref[...], preferred_element_type=jnp.float32)


def kernel(q, kv_src, seg_id, Wq, Wkv, Wo):
    b, lq, qdim = q.shape
    lkv = kv_src.shape[1]
    d = Wq.shape[1]
    nq = lq // BQ

    kvp = pl.pallas_call(
        _kv_proj_body,
        out_shape=jax.ShapeDtypeStruct((b * lkv, 2 * d), jnp.float32),
    )(kv_src.reshape(b * lkv, -1), Wkv)
    k = kvp[:, :d].reshape(b, lkv, d)
    v = kvp[:, d:].reshape(b, lkv, d)

    seg = seg_id.astype(jnp.float32).reshape(b * nq, 1, BQ)
    Wqb = Wq.astype(jnp.bfloat16)
    Wob = Wo.astype(jnp.bfloat16)

    out = pl.pallas_call(
        _attn_body,
        grid=(b, nq),
        in_specs=[
            pl.BlockSpec((1, 1, BQ), lambda bi, i: (bi * (lq // BQ) + i, 0, 0)),
            pl.BlockSpec((1, BQ, qdim), lambda bi, i: (bi, i, 0)),
            pl.BlockSpec((1, lkv, d), lambda bi, i: (bi, 0, 0)),
            pl.BlockSpec((1, lkv, d), lambda bi, i: (bi, 0, 0)),
            pl.BlockSpec((qdim, d), lambda bi, i: (0, 0)),
            pl.BlockSpec((d, qdim), lambda bi, i: (0, 0)),
        ],
        out_specs=pl.BlockSpec((1, BQ, qdim), lambda bi, i: (bi, i, 0)),
        out_shape=jax.ShapeDtypeStruct((b, lq, qdim), jnp.float32),
    )(seg, q, k, v, Wqb, Wob)
    return out


# speedup vs baseline: 1.1916x; 1.0439x over previous
import jax
import jax.numpy as jnp
from jax.experimental import pallas as pl
from jax.experimental.pallas import tpu as pltpu

H = 16
LOOKBACK = 3
BQ = 512


def _kv_proj_body(kv_ref, wkv_ref, out_ref):
    out_ref[...] = jnp.dot(kv_ref[...], wkv_ref[...],
                           preferred_element_type=jnp.float32)


def _attn_body(seg_ref, q_ref, k_ref, v_ref, wq_ref, wo_ref, out_ref):
    qh = jnp.dot(q_ref[0].astype(wq_ref.dtype), wq_ref[...],
                 preferred_element_type=jnp.float32)
    bq, d = qh.shape
    dh = d // H
    lkv = k_ref.shape[1]
    scale = dh ** -0.5
    k = k_ref[0]
    v = v_ref[0]
    seg = seg_ref[0].reshape(bq, 1)
    col = jax.lax.broadcasted_iota(jnp.int32, (bq, lkv), 1).astype(jnp.float32)
    mask = (col <= seg) & (col >= seg - LOOKBACK)
    outs = []
    for h in range(H):
        qh_h = qh[:, h * dh:(h + 1) * dh]
        k_h = k[:, h * dh:(h + 1) * dh]
        v_h = v[:, h * dh:(h + 1) * dh]
        s = jax.lax.dot_general(qh_h, k_h, (((1,), (1,)), ((), ())),
                                preferred_element_type=jnp.float32) * scale
        s = jnp.where(mask, s, -jnp.inf)
        m = jnp.max(s, axis=1, keepdims=True)
        e = jnp.exp(s - m)
        p = e / jnp.sum(e, axis=1, keepdims=True)
        outs.append(jnp.dot(p, v_h, preferred_element_type=jnp.float32))
    attn = jnp.concatenate(outs, axis=1).astype(wo_ref.dtype)
    out_ref[0] = jnp.dot(attn, wo_ref[...], preferred_element_type=jnp.float32)


def kernel(q, kv_src, seg_id, Wq, Wkv, Wo):
    b, lq, qdim = q.shape
    lkv = kv_src.shape[1]
    d = Wq.shape[1]
    nq = lq // BQ

    kvp = pl.pallas_call(
        _kv_proj_body,
        out_shape=jax.ShapeDtypeStruct((b * lkv, 2 * d), jnp.float32),
    )(kv_src.reshape(b * lkv, -1), Wkv)
    k = kvp[:, :d].reshape(b, lkv, d)
    v = kvp[:, d:].reshape(b, lkv, d)

    seg = seg_id.astype(jnp.float32).reshape(b * nq, 1, BQ)
    Wqb = Wq.astype(jnp.bfloat16)
    Wob = Wo.astype(jnp.bfloat16)

    out = pl.pallas_call(
        _attn_body,
        grid=(b, nq),
        in_specs=[
            pl.BlockSpec((1, 1, BQ), lambda bi, i: (bi * (lq // BQ) + i, 0, 0)),
            pl.BlockSpec((1, BQ, qdim), lambda bi, i: (bi, i, 0)),
            pl.BlockSpec((1, lkv, d), lambda bi, i: (bi, 0, 0)),
            pl.BlockSpec((1, lkv, d), lambda bi, i: (bi, 0, 0)),
            pl.BlockSpec((qdim, d), lambda bi, i: (0, 0)),
            pl.BlockSpec((d, qdim), lambda bi, i: (0, 0)),
        ],
        out_specs=pl.BlockSpec((1, BQ, qdim), lambda bi, i: (bi, i, 0)),
        out_shape=jax.ShapeDtypeStruct((b, lq, qdim), jnp.float32),
    )(seg, q, k, v, Wqb, Wob)
    return out


# speedup vs baseline: 1.2176x; 1.0218x over previous
import jax
import jax.numpy as jnp
from jax.experimental import pallas as pl
from jax.experimental.pallas import tpu as pltpu

H = 16
LOOKBACK = 3
BQ = 512


def _kv_proj_body(kv_ref, wkv_ref, out_ref):
    out_ref[...] = jnp.dot(kv_ref[...], wkv_ref[...],
                           preferred_element_type=jnp.float32)


def _attn_body(seg_ref, q_ref, k_ref, v_ref, wq_hbm, wo_hbm, out_ref,
               wtmp_ref, wqb_ref, wob_ref, sem):
    @pl.when((pl.program_id(0) == 0) & (pl.program_id(1) == 0))
    def _():
        pltpu.make_async_copy(wq_hbm, wtmp_ref, sem).start()
        pltpu.make_async_copy(wq_hbm, wtmp_ref, sem).wait()
        wqb_ref[...] = wtmp_ref[...].astype(jnp.bfloat16)
        pltpu.make_async_copy(wo_hbm, wtmp_ref, sem).start()
        pltpu.make_async_copy(wo_hbm, wtmp_ref, sem).wait()
        wob_ref[...] = wtmp_ref[...].astype(jnp.bfloat16)

    qh = jnp.dot(q_ref[0].astype(jnp.bfloat16), wqb_ref[...],
                 preferred_element_type=jnp.float32)
    bq, d = qh.shape
    dh = d // H
    lkv = k_ref.shape[1]
    scale = dh ** -0.5
    k = k_ref[0]
    v = v_ref[0]
    seg = seg_ref[0].reshape(bq, 1)
    col = jax.lax.broadcasted_iota(jnp.int32, (bq, lkv), 1).astype(jnp.float32)
    mask = (col <= seg) & (col >= seg - LOOKBACK)
    outs = []
    for h in range(H):
        qh_h = qh[:, h * dh:(h + 1) * dh]
        k_h = k[:, h * dh:(h + 1) * dh]
        v_h = v[:, h * dh:(h + 1) * dh]
        s = jax.lax.dot_general(qh_h, k_h, (((1,), (1,)), ((), ())),
                                preferred_element_type=jnp.float32) * scale
        s = jnp.where(mask, s, -jnp.inf)
        m = jnp.max(s, axis=1, keepdims=True)
        e = jnp.exp(s - m)
        p = e / jnp.sum(e, axis=1, keepdims=True)
        outs.append(jnp.dot(p, v_h, preferred_element_type=jnp.float32))
    attn = jnp.concatenate(outs, axis=1).astype(jnp.bfloat16)
    out_ref[0] = jnp.dot(attn, wob_ref[...], preferred_element_type=jnp.float32)


def kernel(q, kv_src, seg_id, Wq, Wkv, Wo):
    b, lq, qdim = q.shape
    lkv = kv_src.shape[1]
    d = Wq.shape[1]
    nq = lq // BQ

    kvp = pl.pallas_call(
        _kv_proj_body,
        out_shape=jax.ShapeDtypeStruct((b * lkv, 2 * d), jnp.float32),
    )(kv_src.reshape(b * lkv, -1), Wkv)
    k = kvp[:, :d].reshape(b, lkv, d)
    v = kvp[:, d:].reshape(b, lkv, d)

    seg = seg_id.astype(jnp.float32).reshape(b * nq, 1, BQ)

    out = pl.pallas_call(
        _attn_body,
        grid=(b, nq),
        in_specs=[
            pl.BlockSpec((1, 1, BQ), lambda bi, i: (bi * (lq // BQ) + i, 0, 0)),
            pl.BlockSpec((1, BQ, qdim), lambda bi, i: (bi, i, 0)),
            pl.BlockSpec((1, lkv, d), lambda bi, i: (bi, 0, 0)),
            pl.BlockSpec((1, lkv, d), lambda bi, i: (bi, 0, 0)),
            pl.BlockSpec(memory_space=pl.ANY),
            pl.BlockSpec(memory_space=pl.ANY),
        ],
        out_specs=pl.BlockSpec((1, BQ, qdim), lambda bi, i: (bi, i, 0)),
        out_shape=jax.ShapeDtypeStruct((b, lq, qdim), jnp.float32),
        scratch_shapes=[
            pltpu.VMEM((qdim, d), jnp.float32),
            pltpu.VMEM((qdim, d), jnp.bfloat16),
            pltpu.VMEM((d, qdim), jnp.bfloat16),
            pltpu.SemaphoreType.DMA,
        ],
        compiler_params=pltpu.CompilerParams(
            vmem_limit_bytes=63 * 1024 * 1024,
        ),
    )(seg, q, k, v, Wq, Wo)
    return out
